# Initial kernel scaffold; baseline (speedup 1.0000x reference)
import functools

import jax
import jax.numpy as jnp
from jax import lax
from jax.experimental import pallas as pl
from jax.experimental.pallas import tpu as pltpu
from jax.experimental.pallas import tpu_sc as plsc

VOCAB = 1000000
DIM = 64
BATCH = 16384
CTX = 20
NEG = 20

NC = 2
NS = 16
NW = NC * NS
BPW = BATCH // NW
CB = 32
NCHUNK = BPW // CB
ROWS = CB * CTX
TPG = ROWS // 128
ND = DIM // 16


def _sc_body(ctx_idx_hbm, pos_idx_hbm, neg_idx_hbm, in_emb, out_emb,
             pos_out, neg_out,
             ctx_idx_v, neg_idx_v, pos_idx_v,
             ctx_rows_v, neg_rows_v, pos_rows_v,
             pos_sc_v, neg_sc_v, sem):
    wid = lax.axis_index("s") * NC + lax.axis_index("c")

    pltpu.sync_copy(ctx_idx_hbm.at[pl.ds(pl.multiple_of(wid * BPW * CTX, 8), BPW * CTX)], ctx_idx_v)
    pltpu.sync_copy(neg_idx_hbm.at[pl.ds(pl.multiple_of(wid * BPW * NEG, 8), BPW * NEG)], neg_idx_v)
    pltpu.sync_copy(pos_idx_hbm.at[pl.ds(pl.multiple_of(wid * BPW, 8), BPW)], pos_idx_v)

    def chunk_body(c, carry):
        off = pl.multiple_of(c * ROWS, 8)
        poff = pl.multiple_of(c * CB, 8)
        cps = []
        for t in range(TPG):
            cps.append(pltpu.async_copy(
                in_emb.at[ctx_idx_v.at[pl.ds(off + t * 128, 128)]],
                ctx_rows_v.at[pl.ds(t * 128, 128)], sem))
        for t in range(TPG):
            cps.append(pltpu.async_copy(
                out_emb.at[neg_idx_v.at[pl.ds(off + t * 128, 128)]],
                neg_rows_v.at[pl.ds(t * 128, 128)], sem))
        cps.append(pltpu.async_copy(
            out_emb.at[pos_idx_v.at[pl.ds(poff, CB)]], pos_rows_v, sem))
        for cp in cps:
            cp.wait()

        def elem_body(e, carry2):
            b = c * CB + e
            r0 = e * CTX
            acc = [ctx_rows_v[r0, pl.ds(16 * d, 16)] for d in range(ND)]
            for j in range(1, CTX):
                for d in range(ND):
                    acc[d] = acc[d] + ctx_rows_v[r0 + j, pl.ds(16 * d, 16)]
            cv = [a * (1.0 / CTX) for a in acc]
            pr = [pos_rows_v[e, pl.ds(16 * d, 16)] for d in range(ND)]
            ps = jnp.sum(cv[0] * pr[0] + cv[1] * pr[1] + cv[2] * pr[2] + cv[3] * pr[3])
            pos_sc_v[b] = ps
            for k in range(NEG):
                nr = [neg_rows_v[r0 + k, pl.ds(16 * d, 16)] for d in range(ND)]
                ns = jnp.sum(cv[0] * nr[0] + cv[1] * nr[1] + cv[2] * nr[2] + cv[3] * nr[3])
                neg_sc_v[b * NEG + k] = ns
            return carry2

        lax.fori_loop(0, CB, elem_body, 0)
        return carry

    lax.fori_loop(0, NCHUNK, chunk_body, 0)

    pltpu.sync_copy(pos_sc_v, pos_out.at[pl.ds(pl.multiple_of(wid * BPW, 8), BPW)])
    pltpu.sync_copy(neg_sc_v, neg_out.at[pl.ds(pl.multiple_of(wid * BPW * NEG, 8), BPW * NEG)])


_sc_scores = functools.partial(
    pl.kernel,
    out_type=(jax.ShapeDtypeStruct((BATCH,), jnp.float32),
              jax.ShapeDtypeStruct((BATCH * NEG,), jnp.float32)),
    mesh=plsc.VectorSubcoreMesh(core_axis_name="c", subcore_axis_name="s",
                                num_cores=NC, num_subcores=NS),
    scratch_types=[
        pltpu.VMEM((BPW * CTX,), jnp.int32),
        pltpu.VMEM((BPW * NEG,), jnp.int32),
        pltpu.VMEM((BPW,), jnp.int32),
        pltpu.VMEM((ROWS, DIM), jnp.float32),
        pltpu.VMEM((ROWS, DIM), jnp.float32),
        pltpu.VMEM((CB, DIM), jnp.float32),
        pltpu.VMEM((BPW,), jnp.float32),
        pltpu.VMEM((BPW * NEG,), jnp.float32),
        pltpu.SemaphoreType.DMA,
    ],
)(_sc_body)


def _loss_body(pos_ref, neg_ref, out_ref):
    p = pos_ref[...]
    n = neg_ref[...]
    pls = jnp.log(1.0 / (1.0 + jnp.exp(-p)) + 1e-10)
    nls = jnp.log(1.0 / (1.0 + jnp.exp(n)) + 1e-10)
    out_ref[0, 0] = -(jnp.sum(pls) + jnp.sum(nls)) / BATCH


def kernel(context_idxs, pos_target, neg_samples, in_embed, out_embed):
    ctx_flat = context_idxs.reshape(-1)
    neg_flat = neg_samples.reshape(-1)
    pos_sc, neg_sc = _sc_scores(ctx_flat, pos_target, neg_flat, in_embed, out_embed)
    loss = pl.pallas_call(
        _loss_body,
        out_shape=jax.ShapeDtypeStruct((1, 1), jnp.float32),
    )(pos_sc.reshape(BATCH // 128, 128), neg_sc.reshape(BATCH * NEG // 128, 128))
    return loss[0, 0]


# speedup vs baseline: 4.9973x; 4.9973x over previous
import functools

import jax
import jax.numpy as jnp
from jax import lax
from jax.experimental import pallas as pl
from jax.experimental.pallas import tpu as pltpu
from jax.experimental.pallas import tpu_sc as plsc

VOCAB = 1000000
DIM = 64
BATCH = 16384
CTX = 20
NEG = 20

NC = 2
NS = 16
NW = NC * NS
BPW = BATCH // NW
CB = 32
NCHUNK = BPW // CB
ROWS = CB * CTX
TPG = ROWS // 128
ND = DIM // 16


def _mo8(x):
    return pl.multiple_of(x, 8)


def _sc_body(ctx_idx_hbm, pos_idx_hbm, neg_idx_hbm, in_emb, out_emb,
             pos_out, neg_out,
             ctx_idx_v, neg_idx_v, pos_idx_v,
             ctx_rows_v, neg_rows_v, pos_rows_v,
             pprod_v, nprod_v, pos_sc_v, neg_sc_v, sem):
    wid = lax.axis_index("s") * NC + lax.axis_index("c")

    pltpu.sync_copy(ctx_idx_hbm.at[pl.ds(_mo8(wid * BPW * CTX), BPW * CTX)], ctx_idx_v)
    pltpu.sync_copy(neg_idx_hbm.at[pl.ds(_mo8(wid * BPW * NEG), BPW * NEG)], neg_idx_v)
    pltpu.sync_copy(pos_idx_hbm.at[pl.ds(_mo8(wid * BPW), BPW)], pos_idx_v)

    iota16 = lax.iota(jnp.int32, 16)

    def chunk_body(c, carry):
        off = _mo8(c * ROWS)
        cps = []
        for t in range(TPG):
            cps.append(pltpu.async_copy(
                in_emb.at[ctx_idx_v.at[pl.ds(off + t * 128, 128)]],
                ctx_rows_v.at[pl.ds(t * 128, 128)], sem))
        for t in range(TPG):
            cps.append(pltpu.async_copy(
                out_emb.at[neg_idx_v.at[pl.ds(off + t * 128, 128)]],
                neg_rows_v.at[pl.ds(t * 128, 128)], sem))
        cps.append(pltpu.async_copy(
            out_emb.at[pos_idx_v.at[pl.ds(_mo8(c * CB), CB)]], pos_rows_v, sem))
        for cp in cps:
            cp.wait()

        def elem_body(e, carry2):
            r0 = e * CTX
            acc = [ctx_rows_v[r0, pl.ds(16 * d, 16)] for d in range(ND)]
            for j in range(1, CTX):
                for d in range(ND):
                    acc[d] = acc[d] + ctx_rows_v[r0 + j, pl.ds(16 * d, 16)]
            cv = [a * (1.0 / CTX) for a in acc]
            pr = [pos_rows_v[e, pl.ds(16 * d, 16)] for d in range(ND)]
            pp = cv[0] * pr[0] + cv[1] * pr[1] + cv[2] * pr[2] + cv[3] * pr[3]
            pprod_v[pl.ds(_mo8(e * 16), 16)] = pp
            for k in range(NEG):
                nr = [neg_rows_v[r0 + k, pl.ds(16 * d, 16)] for d in range(ND)]
                np_ = cv[0] * nr[0] + cv[1] * nr[1] + cv[2] * nr[2] + cv[3] * nr[3]
                nprod_v[pl.ds(_mo8((e * NEG + k) * 16), 16)] = np_
            return carry2

        lax.fori_loop(0, CB, elem_body, 0)

        def pgroup(g, carry2):
            base = g * 256
            s = plsc.load_gather(pprod_v, [base + iota16 * 16])
            for d in range(1, 16):
                s = s + plsc.load_gather(pprod_v, [base + iota16 * 16 + d])
            pos_sc_v[pl.ds(_mo8(g * 16), 16)] = s
            return carry2

        lax.fori_loop(0, CB // 16, pgroup, 0)

        def ngroup(g, carry2):
            base = g * 256
            s = plsc.load_gather(nprod_v, [base + iota16 * 16])
            for d in range(1, 16):
                s = s + plsc.load_gather(nprod_v, [base + iota16 * 16 + d])
            neg_sc_v[pl.ds(_mo8(g * 16), 16)] = s
            return carry2

        lax.fori_loop(0, CB * NEG // 16, ngroup, 0)

        pltpu.sync_copy(pos_sc_v, pos_out.at[pl.ds(_mo8(wid * BPW + c * CB), CB)])
        pltpu.sync_copy(neg_sc_v,
                        neg_out.at[pl.ds(_mo8((wid * BPW + c * CB) * NEG), CB * NEG)])
        return carry

    lax.fori_loop(0, NCHUNK, chunk_body, 0)


_sc_scores = functools.partial(
    pl.kernel,
    out_type=(jax.ShapeDtypeStruct((BATCH,), jnp.float32),
              jax.ShapeDtypeStruct((BATCH * NEG,), jnp.float32)),
    mesh=plsc.VectorSubcoreMesh(core_axis_name="c", subcore_axis_name="s",
                                num_cores=NC, num_subcores=NS),
    compiler_params=pltpu.CompilerParams(needs_layout_passes=False,
                                         use_tc_tiling_on_sc=False),
    scratch_types=[
        pltpu.VMEM((BPW * CTX,), jnp.int32),
        pltpu.VMEM((BPW * NEG,), jnp.int32),
        pltpu.VMEM((BPW,), jnp.int32),
        pltpu.VMEM((ROWS, DIM), jnp.float32),
        pltpu.VMEM((ROWS, DIM), jnp.float32),
        pltpu.VMEM((CB, DIM), jnp.float32),
        pltpu.VMEM((CB * 16,), jnp.float32),
        pltpu.VMEM((CB * NEG * 16,), jnp.float32),
        pltpu.VMEM((CB,), jnp.float32),
        pltpu.VMEM((CB * NEG,), jnp.float32),
        pltpu.SemaphoreType.DMA,
    ],
)(_sc_body)


def _loss_body(pos_ref, neg_ref, out_ref):
    p = pos_ref[...]
    n = neg_ref[...]
    pls = jnp.log(1.0 / (1.0 + jnp.exp(-p)) + 1e-10)
    nls = jnp.log(1.0 / (1.0 + jnp.exp(n)) + 1e-10)
    total = -(jnp.sum(pls) + jnp.sum(nls)) / BATCH
    out_ref[...] = jnp.full((1, 1), total, jnp.float32)


def kernel(context_idxs, pos_target, neg_samples, in_embed, out_embed):
    ctx_flat = context_idxs.reshape(-1)
    neg_flat = neg_samples.reshape(-1)
    pos_sc, neg_sc = _sc_scores(ctx_flat, pos_target, neg_flat, in_embed, out_embed)
    loss = pl.pallas_call(
        _loss_body,
        out_shape=jax.ShapeDtypeStruct((1, 1), jnp.float32),
    )(pos_sc.reshape(BATCH // 128, 128), neg_sc.reshape(BATCH * NEG // 128, 128))
    return loss[0, 0]
